# baseline (speedup 1.0000x reference)
import jax
import jax.numpy as jnp
from jax.experimental import pallas as pl
from jax.experimental.pallas import tpu as pltpu


def _scale_kernel(x_ref, o_ref):
    o_ref[...] = x_ref[...] * 0.9


def kernel(x, W, b):
    del W, b
    B, F = x.shape
    blk = 1024
    return pl.pallas_call(
        _scale_kernel,
        grid=(B // blk,),
        in_specs=[pl.BlockSpec((blk, F), lambda i: (i, 0))],
        out_specs=pl.BlockSpec((blk, F), lambda i: (i, 0)),
        out_shape=jax.ShapeDtypeStruct((B, F), x.dtype),
        compiler_params=pltpu.CompilerParams(
            dimension_semantics=("parallel",),
        ),
    )(x)


# speedup vs baseline: 1.4353x; 1.4353x over previous
import jax
import jax.numpy as jnp
from jax.experimental import pallas as pl
from jax.experimental.pallas import tpu as pltpu


def _scale_kernel(x_ref, o_ref):
    o_ref[...] = x_ref[...] * 0.9


def kernel(x, W, b):
    del W, b
    B, F = x.shape
    blk = 4096
    return pl.pallas_call(
        _scale_kernel,
        grid=(B // blk,),
        in_specs=[pl.BlockSpec((blk, F), lambda i: (i, 0))],
        out_specs=pl.BlockSpec((blk, F), lambda i: (i, 0)),
        out_shape=jax.ShapeDtypeStruct((B, F), x.dtype),
    )(x)


# speedup vs baseline: 1.6115x; 1.1228x over previous
import jax
import jax.numpy as jnp
from jax.experimental import pallas as pl
from jax.experimental.pallas import tpu as pltpu


def _scale_kernel(x_ref, o_ref):
    o_ref[...] = x_ref[...] * 0.9


def kernel(x, W, b):
    del W, b
    B, F = x.shape
    blk = 8192
    return pl.pallas_call(
        _scale_kernel,
        grid=(B // blk,),
        in_specs=[pl.BlockSpec((blk, F), lambda i: (i, 0))],
        out_specs=pl.BlockSpec((blk, F), lambda i: (i, 0)),
        out_shape=jax.ShapeDtypeStruct((B, F), x.dtype),
    )(x)
